# initial kernel scaffold (unmeasured)
import jax
import jax.numpy as jnp
from jax import lax
from jax.experimental import pallas as pl
from jax.experimental.pallas import tpu as pltpu

N_DEV = 8
M_BLK = 512
K_BLK = 512
N_OUT = 2048


def kernel(x, w_mat, scale_x, scale_w):
    m_total, k_shard = x.shape
    assert m_total == N_DEV * M_BLK and k_shard == K_BLK, x.shape

    def body(x_ref, w_ref, sx_ref, sw_ref, out_ref, buf, send_sems, recv_sems):
        me = lax.axis_index("i")

        barrier = pltpu.get_barrier_semaphore()
        for s in range(1, N_DEV):
            peer = lax.rem(me + s, N_DEV)
            pl.semaphore_signal(
                barrier, inc=1,
                device_id=(peer,), device_id_type=pl.DeviceIdType.MESH,
            )
        pl.semaphore_wait(barrier, N_DEV - 1)

        sends = []
        for s in range(1, N_DEV):
            dst = lax.rem(me + s, N_DEV)
            rdma = pltpu.make_async_remote_copy(
                src_ref=x_ref.at[pl.ds(dst * M_BLK, M_BLK), :],
                dst_ref=buf.at[me],
                send_sem=send_sems.at[s],
                recv_sem=recv_sems.at[s],
                device_id=(dst,),
                device_id_type=pl.DeviceIdType.MESH,
            )
            rdma.start()
            sends.append(rdma)

        scale = sx_ref[0] * sw_ref[0]

        def mm(a, j):
            wj = w_ref[pl.ds(j * K_BLK, K_BLK), :]
            return lax.dot_general(
                a, wj, (((1,), (0,)), ((), ())),
                preferred_element_type=jnp.float32,
            )

        acc = mm(x_ref[pl.ds(me * M_BLK, M_BLK), :], me)

        for s in range(1, N_DEV):
            src_dev = lax.rem(me - s + N_DEV, N_DEV)
            recv = pltpu.make_async_remote_copy(
                src_ref=x_ref.at[pl.ds(0, M_BLK), :],
                dst_ref=buf.at[src_dev],
                send_sem=send_sems.at[s],
                recv_sem=recv_sems.at[s],
                device_id=(me,),
                device_id_type=pl.DeviceIdType.MESH,
            )
            recv.wait_recv()
            acc += mm(buf[src_dev], src_dev)

        out_ref[:, :] = jnp.maximum(acc * scale, 0.0)

        for rdma in sends:
            rdma.wait_send()

    return pl.pallas_call(
        body,
        out_shape=jax.ShapeDtypeStruct((M_BLK, N_OUT), jnp.float32),
        in_specs=[
            pl.BlockSpec(memory_space=pltpu.VMEM),
            pl.BlockSpec(memory_space=pltpu.VMEM),
            pl.BlockSpec(memory_space=pltpu.SMEM),
            pl.BlockSpec(memory_space=pltpu.SMEM),
        ],
        out_specs=pl.BlockSpec(memory_space=pltpu.VMEM),
        scratch_shapes=[
            pltpu.VMEM((N_DEV, M_BLK, K_BLK), x.dtype),
            pltpu.SemaphoreType.DMA((N_DEV,)),
            pltpu.SemaphoreType.DMA((N_DEV,)),
        ],
        compiler_params=pltpu.CompilerParams(collective_id=0),
    )(x, w_mat, scale_x, scale_w)


# baseline (device time: 34381 ns/iter reference)
import jax
import jax.numpy as jnp
from jax import lax
from jax.experimental import pallas as pl
from jax.experimental.pallas import tpu as pltpu

N_DEV = 8
M_BLK = 512
K_BLK = 512
N_OUT = 2048
FP8 = jnp.float8_e5m2


def kernel(x, w_mat, scale_x, scale_w):
    m_total, k_shard = x.shape
    assert m_total == N_DEV * M_BLK and k_shard == K_BLK, x.shape

    def body(x_ref, w_ref, sx_ref, sw_ref, out_ref,
             xq, buf, wbuf, send_sems, recv_sems, w_sems):
        me = lax.axis_index("i")

        def k_block(t):
            return me if t == 0 else lax.rem(me - t + N_DEV, N_DEV)

        def w_copy(t, slot):
            j = k_block(t)
            return pltpu.make_async_copy(
                w_ref.at[pl.ds(j * K_BLK, K_BLK), :],
                wbuf.at[slot],
                w_sems.at[slot],
            )

        w_copy(0, 0).start()
        w_copy(1, 1).start()

        for p in range(N_DEV):
            rows = pl.ds(p * M_BLK, M_BLK)
            xq[rows, :] = x_ref[rows, :].astype(FP8)

        barrier = pltpu.get_barrier_semaphore()
        for s in range(1, N_DEV):
            peer = lax.rem(me + s, N_DEV)
            pl.semaphore_signal(
                barrier, inc=1,
                device_id=(peer,), device_id_type=pl.DeviceIdType.MESH,
            )
        pl.semaphore_wait(barrier, N_DEV - 1)

        sends = []
        for s in range(1, N_DEV):
            dst = lax.rem(me + s, N_DEV)
            rdma = pltpu.make_async_remote_copy(
                src_ref=xq.at[pl.ds(dst * M_BLK, M_BLK), :],
                dst_ref=buf.at[me],
                send_sem=send_sems.at[s],
                recv_sem=recv_sems.at[s],
                device_id=(dst,),
                device_id_type=pl.DeviceIdType.MESH,
            )
            rdma.start()
            sends.append(rdma)

        scale = sx_ref[0] * sw_ref[0]

        acc = None
        for t in range(N_DEV):
            slot = t % 2
            w_copy(t, slot).wait()
            if t == 0:
                a = xq[pl.ds(me * M_BLK, M_BLK), :]
            else:
                src_dev = k_block(t)
                recv = pltpu.make_async_remote_copy(
                    src_ref=xq.at[pl.ds(0, M_BLK), :],
                    dst_ref=buf.at[src_dev],
                    send_sem=send_sems.at[t],
                    recv_sem=recv_sems.at[t],
                    device_id=(me,),
                    device_id_type=pl.DeviceIdType.MESH,
                )
                recv.wait_recv()
                a = buf[src_dev]
            contrib = lax.dot_general(
                a, wbuf[slot].astype(FP8), (((1,), (0,)), ((), ())),
                preferred_element_type=jnp.float32,
            )
            acc = contrib if acc is None else acc + contrib
            if t + 2 < N_DEV:
                w_copy(t + 2, slot).start()

        out_ref[:, :] = jnp.maximum(acc * scale, 0.0)

        for rdma in sends:
            rdma.wait_send()

    return pl.pallas_call(
        body,
        out_shape=jax.ShapeDtypeStruct((M_BLK, N_OUT), jnp.float32),
        in_specs=[
            pl.BlockSpec(memory_space=pltpu.VMEM),
            pl.BlockSpec(memory_space=pl.ANY),
            pl.BlockSpec(memory_space=pltpu.SMEM),
            pl.BlockSpec(memory_space=pltpu.SMEM),
        ],
        out_specs=pl.BlockSpec(memory_space=pltpu.VMEM),
        scratch_shapes=[
            pltpu.VMEM((N_DEV * M_BLK, K_BLK), FP8),
            pltpu.VMEM((N_DEV, M_BLK, K_BLK), FP8),
            pltpu.VMEM((2, K_BLK, N_OUT), jnp.float32),
            pltpu.SemaphoreType.DMA((N_DEV,)),
            pltpu.SemaphoreType.DMA((N_DEV,)),
            pltpu.SemaphoreType.DMA((2,)),
        ],
        compiler_params=pltpu.CompilerParams(
            collective_id=0,
            vmem_limit_bytes=63 * 1024 * 1024,
        ),
    )(x, w_mat, scale_x, scale_w)


# device time: 33152 ns/iter; 1.0371x vs baseline; 1.0371x over previous
import jax
import jax.numpy as jnp
from jax import lax
from jax.experimental import pallas as pl
from jax.experimental.pallas import tpu as pltpu

N_DEV = 8
M_BLK = 512
K_BLK = 512
N_OUT = 2048
FP8 = jnp.float8_e5m2


def kernel(x, w_mat, scale_x, scale_w):
    m_total, k_shard = x.shape
    assert m_total == N_DEV * M_BLK and k_shard == K_BLK, x.shape

    def body(x_ref, w_ref, sx_ref, sw_ref, out_ref,
             xf, xq, buf, wbuf, send_sems, recv_sems, x_sems, w_sems):
        me = lax.axis_index("i")

        def k_block(t):
            return me if t == 0 else lax.rem(me - t + N_DEV, N_DEV)

        def w_copy(t, slot):
            j = k_block(t)
            return pltpu.make_async_copy(
                w_ref.at[pl.ds(j * K_BLK, K_BLK), :],
                wbuf.at[slot], w_sems.at[slot])

        def x_stage(p, slot):
            return pltpu.make_async_copy(
                x_ref.at[pl.ds(p * M_BLK, M_BLK), :],
                xf.at[slot], x_sems.at[slot])

        w_copy(0, 0).start()
        w_copy(1, 1).start()

        barrier = pltpu.get_barrier_semaphore()
        for s in range(1, N_DEV):
            peer = lax.rem(me + s, N_DEV)
            pl.semaphore_signal(
                barrier, inc=1,
                device_id=(peer,), device_id_type=pl.DeviceIdType.MESH,
            )
        pl.semaphore_wait(barrier, N_DEV - 1)

        stage_order = [lax.rem(me + s, N_DEV) for s in range(1, N_DEV)] + [me]
        x_stage(stage_order[0], 0).start()
        x_stage(stage_order[1], 1).start()
        sends = []
        for idx, p in enumerate(stage_order):
            slot = idx % 2
            x_stage(p, slot).wait()
            xq[p] = xf[slot].astype(FP8)
            if idx + 2 < N_DEV:
                x_stage(stage_order[idx + 2], slot).start()
            if idx < N_DEV - 1:
                s = idx + 1
                rdma = pltpu.make_async_remote_copy(
                    src_ref=xq.at[p],
                    dst_ref=buf.at[me],
                    send_sem=send_sems.at[s],
                    recv_sem=recv_sems.at[s],
                    device_id=(p,),
                    device_id_type=pl.DeviceIdType.MESH,
                )
                rdma.start()
                sends.append(rdma)

        scale = sx_ref[0] * sw_ref[0]

        acc = None
        for t in range(N_DEV):
            slot = t % 2
            w_copy(t, slot).wait()
            if t == 0:
                a = xq[me]
            else:
                src_dev = k_block(t)
                recv = pltpu.make_async_remote_copy(
                    src_ref=xq.at[0],
                    dst_ref=buf.at[src_dev],
                    send_sem=send_sems.at[t],
                    recv_sem=recv_sems.at[t],
                    device_id=(me,),
                    device_id_type=pl.DeviceIdType.MESH,
                )
                recv.wait_recv()
                a = buf[src_dev]
            contrib = lax.dot_general(
                a, wbuf[slot].astype(FP8), (((1,), (0,)), ((), ())),
                preferred_element_type=jnp.float32,
            )
            acc = contrib if acc is None else acc + contrib
            if t + 2 < N_DEV:
                w_copy(t + 2, slot).start()

        out_ref[:, :] = jnp.maximum(acc * scale, 0.0)

        for rdma in sends:
            rdma.wait_send()

    return pl.pallas_call(
        body,
        out_shape=jax.ShapeDtypeStruct((M_BLK, N_OUT), jnp.float32),
        in_specs=[
            pl.BlockSpec(memory_space=pl.ANY),
            pl.BlockSpec(memory_space=pl.ANY),
            pl.BlockSpec(memory_space=pltpu.SMEM),
            pl.BlockSpec(memory_space=pltpu.SMEM),
        ],
        out_specs=pl.BlockSpec(memory_space=pltpu.VMEM),
        scratch_shapes=[
            pltpu.VMEM((2, M_BLK, K_BLK), jnp.float32),
            pltpu.VMEM((N_DEV, M_BLK, K_BLK), FP8),
            pltpu.VMEM((N_DEV, M_BLK, K_BLK), FP8),
            pltpu.VMEM((2, K_BLK, N_OUT), jnp.float32),
            pltpu.SemaphoreType.DMA((N_DEV,)),
            pltpu.SemaphoreType.DMA((N_DEV,)),
            pltpu.SemaphoreType.DMA((2,)),
            pltpu.SemaphoreType.DMA((2,)),
        ],
        compiler_params=pltpu.CompilerParams(
            collective_id=0,
            vmem_limit_bytes=63 * 1024 * 1024,
        ),
    )(x, w_mat, scale_x, scale_w)


# device time: 31438 ns/iter; 1.0936x vs baseline; 1.0545x over previous
import jax
import jax.numpy as jnp
from jax import lax
from jax.experimental import pallas as pl
from jax.experimental.pallas import tpu as pltpu

N_DEV = 8
M_BLK = 512
K_BLK = 512
N_OUT = 2048
FP8 = jnp.float8_e5m2


def kernel(x, w_mat, scale_x, scale_w):
    m_total, k_shard = x.shape
    assert m_total == N_DEV * M_BLK and k_shard == K_BLK, x.shape

    def body(x_ref, w_ref, sx_ref, sw_ref, out_ref,
             xf, xq, buf, wbuf, wq, vout,
             send_sems, recv_sems, x_sems, w_sems, out_sem):
        me = lax.axis_index("i")

        barrier = pltpu.get_barrier_semaphore()
        for s in range(1, N_DEV):
            peer = lax.rem(me + s, N_DEV)
            pl.semaphore_signal(
                barrier, inc=1,
                device_id=(peer,), device_id_type=pl.DeviceIdType.MESH,
            )

        def k_block(t):
            return me if t == 0 else lax.rem(me - t + N_DEV, N_DEV)

        def w_copy(t, slot):
            j = k_block(t)
            return pltpu.make_async_copy(
                w_ref.at[pl.ds(j * K_BLK, K_BLK), :],
                wbuf.at[slot], w_sems.at[slot])

        def x_stage(p, slot):
            return pltpu.make_async_copy(
                x_ref.at[pl.ds(p * M_BLK, M_BLK), :],
                xf.at[slot], x_sems.at[slot])

        w_copy(0, 0).start()
        w_copy(1, 1).start()

        stage_order = [lax.rem(me + s, N_DEV) for s in range(1, N_DEV)] + [me]
        x_stage(stage_order[0], 0).start()
        x_stage(stage_order[1], 1).start()
        sends = []
        for idx, p in enumerate(stage_order):
            slot = idx % 2
            x_stage(p, slot).wait()
            xq[p] = xf[slot].astype(FP8)
            if idx + 2 < N_DEV:
                x_stage(stage_order[idx + 2], slot).start()
            if idx == 0:
                pl.semaphore_wait(barrier, N_DEV - 1)
            if idx < N_DEV - 1:
                s = idx + 1
                rdma = pltpu.make_async_remote_copy(
                    src_ref=xq.at[p],
                    dst_ref=buf.at[me],
                    send_sem=send_sems.at[s],
                    recv_sem=recv_sems.at[s],
                    device_id=(p,),
                    device_id_type=pl.DeviceIdType.MESH,
                )
                rdma.start()
                sends.append(rdma)

        for t in range(N_DEV):
            slot = t % 2
            w_copy(t, slot).wait()
            j = k_block(t)
            wq[j] = wbuf[slot].astype(FP8)
            if t + 2 < N_DEV:
                w_copy(t + 2, slot).start()

        def mm(a, j):
            return lax.dot_general(
                a, wq[j], (((1,), (0,)), ((), ())),
                preferred_element_type=jnp.float32,
            )

        acc = mm(xq[me], me)
        for t in range(1, N_DEV):
            src_dev = k_block(t)
            recv = pltpu.make_async_remote_copy(
                src_ref=xq.at[0],
                dst_ref=buf.at[src_dev],
                send_sem=send_sems.at[t],
                recv_sem=recv_sems.at[t],
                device_id=(me,),
                device_id_type=pl.DeviceIdType.MESH,
            )
            recv.wait_recv()
            acc += mm(buf[src_dev], src_dev)

        vout[:, :] = jnp.maximum(acc * (sx_ref[0] * sw_ref[0]), 0.0)
        out_cp = pltpu.make_async_copy(vout, out_ref, out_sem)
        out_cp.start()
        out_cp.wait()

        for rdma in sends:
            rdma.wait_send()

    return pl.pallas_call(
        body,
        out_shape=jax.ShapeDtypeStruct((M_BLK, N_OUT), jnp.float32),
        in_specs=[
            pl.BlockSpec(memory_space=pl.ANY),
            pl.BlockSpec(memory_space=pl.ANY),
            pl.BlockSpec(memory_space=pltpu.SMEM),
            pl.BlockSpec(memory_space=pltpu.SMEM),
        ],
        out_specs=pl.BlockSpec(memory_space=pl.ANY),
        scratch_shapes=[
            pltpu.VMEM((2, M_BLK, K_BLK), jnp.float32),
            pltpu.VMEM((N_DEV, M_BLK, K_BLK), FP8),
            pltpu.VMEM((N_DEV, M_BLK, K_BLK), FP8),
            pltpu.VMEM((2, K_BLK, N_OUT), jnp.float32),
            pltpu.VMEM((N_DEV, K_BLK, N_OUT), FP8),
            pltpu.VMEM((M_BLK, N_OUT), jnp.float32),
            pltpu.SemaphoreType.DMA((N_DEV,)),
            pltpu.SemaphoreType.DMA((N_DEV,)),
            pltpu.SemaphoreType.DMA((2,)),
            pltpu.SemaphoreType.DMA((2,)),
            pltpu.SemaphoreType.DMA(()),
        ],
        compiler_params=pltpu.CompilerParams(
            collective_id=0,
            vmem_limit_bytes=63 * 1024 * 1024,
        ),
    )(x, w_mat, scale_x, scale_w)
